# baseline (speedup 1.0000x reference)
import functools

import jax
import jax.numpy as jnp
from jax import lax
from jax.experimental import pallas as pl
from jax.experimental.pallas import tpu as pltpu

_VMEM_LIMIT_BYTES = 48 * 1024 * 1024


def _split_cast_kernel(x_ref, o_ref, *, w):
    x = x_ref[0]
    o_ref[0, 0] = x[:, :, :w].astype(jnp.bfloat16)
    o_ref[0, 1] = x[:, :, w:].astype(jnp.bfloat16)


def _unpack_cols(packed_i32):
    even = lax.bitcast_convert_type(
        packed_i32.astype(jnp.int16), jnp.bfloat16)
    odd = lax.bitcast_convert_type(
        lax.shift_right_logical(packed_i32, jnp.int32(16)).astype(jnp.int16),
        jnp.bfloat16)
    return even, odd


def _conv_kernel(x_ref, w_ref, b_ref, o_ref, *, wout, m):
    cin = x_ref.shape[2]
    ee, eo = _unpack_cols(x_ref[0, 0])
    oe, oo = _unpack_cols(x_ref[0, 1])

    lane = lax.broadcasted_iota(jnp.int32, (1, m), 1)
    col0 = (lane % wout) == 0

    def shift_right(a, s):
        return jnp.concatenate(
            [jnp.zeros((cin, s), a.dtype), a[:, :m - s]], axis=-1)

    def mask_col0(a):
        return jnp.where(col0, jnp.zeros((), a.dtype), a)

    taps = (
        mask_col0(shift_right(oo, wout + 1)),
        shift_right(oe, wout),
        shift_right(oo, wout),
        mask_col0(shift_right(eo, 1)),
        ee,
        eo,
        mask_col0(shift_right(oo, 1)),
        oe,
        oo,
    )

    cout = b_ref.shape[0]
    acc = jnp.dot(w_ref[0:cout, :], taps[0],
                  preferred_element_type=jnp.float32)
    for t in range(1, 9):
        acc += jnp.dot(w_ref[t * cout:(t + 1) * cout, :], taps[t],
                       preferred_element_type=jnp.float32)
    o_ref[0] = acc + b_ref[...]


def kernel(x_nchw, w_oihw, bias):
    n, cin, h, w = x_nchw.shape
    cout = w_oihw.shape[0]
    hout, wout = h // 2, w // 2
    m = hout * wout

    xr = x_nchw.reshape(n, cin, hout, 2 * w)

    x_split = pl.pallas_call(
        functools.partial(_split_cast_kernel, w=w),
        out_shape=jax.ShapeDtypeStruct((n, 2, cin, hout, w), jnp.bfloat16),
        grid=(n,),
        in_specs=[pl.BlockSpec((1, cin, hout, 2 * w), lambda i: (i, 0, 0, 0))],
        out_specs=pl.BlockSpec((1, 2, cin, hout, w), lambda i: (i, 0, 0, 0, 0)),
        compiler_params=pltpu.CompilerParams(
            dimension_semantics=("parallel",),
            vmem_limit_bytes=_VMEM_LIMIT_BYTES),
        cost_estimate=pl.CostEstimate(
            flops=0, transcendentals=0, bytes_accessed=xr.size * 6),
    )(xr)

    x_i32 = lax.bitcast_convert_type(
        x_split.reshape(n, 2, cin, hout, wout, 2),
        jnp.int32).reshape(n, 2, cin, m)

    w2 = jnp.transpose(w_oihw, (2, 3, 0, 1)).reshape(9 * cout, cin)
    w2 = w2.astype(jnp.bfloat16)
    b2 = bias.astype(jnp.float32).reshape(cout, 1)

    cost = pl.CostEstimate(
        flops=2 * n * m * 9 * cin * cout,
        transcendentals=0,
        bytes_accessed=x_i32.size * 4 + w2.size * 2 + n * cout * m * 4,
    )

    out = pl.pallas_call(
        functools.partial(_conv_kernel, wout=wout, m=m),
        out_shape=jax.ShapeDtypeStruct((n, cout, m), jnp.float32),
        grid=(n,),
        in_specs=[
            pl.BlockSpec((1, 2, cin, m), lambda i: (i, 0, 0, 0)),
            pl.BlockSpec((9 * cout, cin), lambda i: (0, 0)),
            pl.BlockSpec((cout, 1), lambda i: (0, 0)),
        ],
        out_specs=pl.BlockSpec((1, cout, m), lambda i: (i, 0, 0)),
        compiler_params=pltpu.CompilerParams(
            dimension_semantics=("parallel",),
            vmem_limit_bytes=_VMEM_LIMIT_BYTES),
        cost_estimate=cost,
    )(x_i32, w2, b2)

    return out.reshape(n, cout, hout, wout).astype(x_nchw.dtype)
